# baseline (speedup 1.0000x reference)
import functools

import jax
import jax.numpy as jnp
from jax import lax
from jax.experimental import pallas as pl
from jax.experimental.pallas import tpu as pltpu
from jax.experimental.pallas import tpu_sc as plsc

N = 9996
E = 319872
NG = 147
NN = 68
H = 128
C = 16
EPS = 1e-15

NP = 10240
PADROW = NP - 1
NW = 32
K = 96
EW = (E + NW * K - 1) // (NW * K) * K
NCH = EW // K
NCHL = ((NCH + 1) // 2) * 2
CHT = NCHL + 2
ZR = NP // 16


def _sc_segsum(D):
    mesh = plsc.VectorSubcoreMesh(core_axis_name="c", subcore_axis_name="s")

    @functools.partial(
        pl.kernel,
        out_type=jax.ShapeDtypeStruct((2, NP, D), jnp.float32),
        mesh=mesh,
        compiler_params=pltpu.CompilerParams(use_tc_tiling_on_sc=False),
        scratch_types=[
            pltpu.VMEM((CHT, K), jnp.int32),
            pltpu.VMEM((CHT, K), jnp.int32),
            pltpu.VMEM((2, K, D), jnp.float32),
            pltpu.VMEM_SHARED((NP, D), jnp.float32),
            [pltpu.SemaphoreType.DMA] * 2,
        ],
    )
    def k(h_hbm, srcr_hbm, dstr_hbm, zeros_hbm, out_hbm,
          sidx, didx, rows_v, acc_sh, gsems):
        c = lax.axis_index("c")
        sid = lax.axis_index("s")
        w = sid * 2 + c
        pltpu.sync_copy(srcr_hbm.at[w], sidx)
        pltpu.sync_copy(dstr_hbm.at[w], didx)
        pltpu.sync_copy(zeros_hbm.at[pl.ds(sid * ZR, ZR)],
                        acc_sh.at[pl.ds(sid * ZR, ZR)])
        plsc.subcore_barrier()

        pltpu.async_copy(h_hbm.at[sidx.at[0]], rows_v.at[0], gsems[0])

        @pl.loop(0, NCHL, step=2)
        def _(base):
            for bb in range(2):
                j = base + bb
                b = bb
                pltpu.make_async_copy(
                    h_hbm.at[sidx.at[j]], rows_v.at[b], gsems[b]).wait()
                pltpu.async_copy(h_hbm.at[sidx.at[j + 1]],
                                 rows_v.at[1 - b], gsems[1 - b])
                pltpu.sync_copy(rows_v.at[b],
                                acc_sh.at[didx.at[j]], add=True)

        pltpu.make_async_copy(
            h_hbm.at[sidx.at[NCHL]], rows_v.at[0], gsems[0]).wait()

        plsc.subcore_barrier()
        pltpu.sync_copy(acc_sh.at[pl.ds(sid * ZR, ZR)],
                        out_hbm.at[c, pl.ds(sid * ZR, ZR)])

    return k


def _tc_layer(p, h_prev, A, Br, bias):
    Dp = p.shape[2]
    Din = h_prev.shape[1]
    BRW = 512

    def body(p_ref, h_ref, a_ref, b_ref, bias_ref, o_ref):
        agg = p_ref[0] + p_ref[1]
        acc = jnp.dot(agg, a_ref[...], preferred_element_type=jnp.float32)
        acc = acc + jnp.dot(h_ref[...], b_ref[...],
                            preferred_element_type=jnp.float32)
        acc = acc + bias_ref[...]
        i = pl.program_id(0)
        rows = i * BRW + lax.broadcasted_iota(jnp.int32, (BRW, 1), 0)
        o_ref[...] = jnp.where(rows < N, jnp.maximum(acc, 0.0), 0.0)

    return pl.pallas_call(
        body,
        grid=(NP // BRW,),
        in_specs=[
            pl.BlockSpec((2, BRW, Dp), lambda i: (0, i, 0)),
            pl.BlockSpec((BRW, Din), lambda i: (i, 0)),
            pl.BlockSpec((Dp, H), lambda i: (0, 0)),
            pl.BlockSpec((Din, H), lambda i: (0, 0)),
            pl.BlockSpec((1, H), lambda i: (0, 0)),
        ],
        out_specs=pl.BlockSpec((BRW, H), lambda i: (i, 0)),
        out_shape=jax.ShapeDtypeStruct((NP, H), jnp.float32),
    )(p, h_prev, A, Br, bias)


BG = 7


def _tc_pool(xr, adj_p, s_p):
    def body(xr_ref, adj_ref, s_ref, out_ref, oadj_ref, acc_ref):
        g = pl.program_id(0)

        @pl.when(g == 0)
        def _():
            acc_ref[0, 0] = 0.0
            acc_ref[0, 1] = 0.0

        link_tot = jnp.float32(0.0)
        ent_tot = jnp.float32(0.0)
        rows = lax.broadcasted_iota(jnp.int32, (128, 1), 0)
        for t in range(BG):
            sg = s_ref[t]
            m = jnp.exp(sg - jnp.max(sg, axis=-1, keepdims=True))
            ssm = m / jnp.sum(m, axis=-1, keepdims=True)
            ssm = jnp.where(rows < NN, ssm, 0.0)
            xg = xr_ref[t]
            ag = adj_ref[t]
            out_ref[t] = lax.dot_general(
                ssm, xg, (((0,), (0,)), ((), ())),
                preferred_element_type=jnp.float32)
            ta = lax.dot_general(
                ssm, ag, (((0,), (0,)), ((), ())),
                preferred_element_type=jnp.float32)
            oadj_ref[t] = lax.dot_general(
                ta, ssm, (((1,), (0,)), ((), ())),
                preferred_element_type=jnp.float32)
            link = ag - lax.dot_general(
                ssm, ssm, (((1,), (1,)), ((), ())),
                preferred_element_type=jnp.float32)
            link_tot = link_tot + jnp.sum(link * link)
            ent_tot = ent_tot + jnp.sum(-ssm * jnp.log(ssm + EPS))
        acc_ref[0, 0] += link_tot
        acc_ref[0, 1] += ent_tot

    return pl.pallas_call(
        body,
        grid=(NG // BG,),
        in_specs=[
            pl.BlockSpec((BG, 128, H), lambda g: (g, 0, 0)),
            pl.BlockSpec((BG, 128, 128), lambda g: (g, 0, 0)),
            pl.BlockSpec((BG, 128, C), lambda g: (g, 0, 0)),
        ],
        out_specs=[
            pl.BlockSpec((BG, C, H), lambda g: (g, 0, 0)),
            pl.BlockSpec((BG, C, C), lambda g: (g, 0, 0)),
            pl.BlockSpec(memory_space=pltpu.SMEM),
        ],
        out_shape=[
            jax.ShapeDtypeStruct((NG, C, H), jnp.float32),
            jax.ShapeDtypeStruct((NG, C, C), jnp.float32),
            jax.ShapeDtypeStruct((1, 2), jnp.float32),
        ],
    )(xr, adj_p, s_p)


def kernel(x, edge_index, adj, s, pos,
           W_rel1, b_rel1, W_root1,
           W_rel2, b_rel2, W_root2,
           W_rel3, b_rel3, W_root3):
    src = edge_index[0]
    dst = edge_index[1]
    padn = NW * NCH * K - E
    dummy = jnp.full((NW, CHT - NCH, K), PADROW, jnp.int32)

    def _reshape_idx(v):
        vr = jnp.concatenate(
            [v, jnp.full((padn,), PADROW, jnp.int32)]).reshape(NW, NCH, K)
        return jnp.concatenate([vr, dummy], axis=1)

    srcr = _reshape_idx(src)
    dstr = _reshape_idx(dst)

    x16 = jnp.zeros((NP, 16), jnp.float32).at[:N, :2].set(x)
    z16 = jnp.zeros((NP, 16), jnp.float32)
    z128 = jnp.zeros((NP, H), jnp.float32)

    A1 = jnp.zeros((16, H), jnp.float32).at[:2, :].set(W_rel1.T)
    B1 = jnp.zeros((16, H), jnp.float32).at[:2, :].set(W_root1.T)

    seg16 = _sc_segsum(16)
    seg128 = _sc_segsum(H)

    p1 = seg16(x16, srcr, dstr, z16)
    h1 = _tc_layer(p1, x16, A1, B1, b_rel1.reshape(1, H))

    p2 = seg128(h1, srcr, dstr, z128)
    h2 = _tc_layer(p2, h1, W_rel2.T, W_root2.T, b_rel2.reshape(1, H))

    p3 = seg128(h2, srcr, dstr, z128)
    h3 = _tc_layer(p3, h2, W_rel3.T, W_root3.T, b_rel3.reshape(1, H))

    xr = h3[:N].reshape(NG, NN, H)
    xr_p = jnp.zeros((NG, 128, H), jnp.float32).at[:, :NN, :].set(xr)
    adj_p = jnp.zeros((NG, 128, 128), jnp.float32).at[:, :NN, :NN].set(adj)
    s_p = jnp.zeros((NG, 128, C), jnp.float32).at[:, :NN, :].set(s)

    out, out_adj, acc = _tc_pool(xr_p, adj_p, s_p)
    link_loss = jnp.sqrt(acc[0, 0]) / (NG * NN * NN)
    ent_loss = acc[0, 1] / (NG * NN)
    return out, out_adj, link_loss, ent_loss, pos


# speedup vs baseline: 1.4195x; 1.4195x over previous
import functools

import jax
import jax.numpy as jnp
from jax import lax
from jax.experimental import pallas as pl
from jax.experimental.pallas import tpu as pltpu
from jax.experimental.pallas import tpu_sc as plsc

N = 9996
E = 319872
NG = 147
NN = 68
H = 128
C = 16
EPS = 1e-15

NP = 10240
PADROW = NP - 1
NW = 32
K = 128
EW = (E + NW * K - 1) // (NW * K) * K
NCH = EW // K
NCHL = NCH
CHT = NCH
ZR = NP // 16


def _sc_segsum(D):
    mesh = plsc.VectorSubcoreMesh(core_axis_name="c", subcore_axis_name="s")

    @functools.partial(
        pl.kernel,
        out_type=jax.ShapeDtypeStruct((2, NP, D), jnp.float32),
        mesh=mesh,
        compiler_params=pltpu.CompilerParams(use_tc_tiling_on_sc=False),
        scratch_types=[
            pltpu.VMEM((CHT, K), jnp.int32),
            pltpu.VMEM((CHT, K), jnp.int32),
            pltpu.VMEM((K, D), jnp.float32),
            pltpu.VMEM_SHARED((NP, D), jnp.float32),
            pltpu.SemaphoreType.DMA,
        ],
    )
    def k(h_hbm, srcr_hbm, dstr_hbm, zeros_hbm, out_hbm,
          sidx, didx, rows_v, acc_sh, gsem):
        c = lax.axis_index("c")
        sid = lax.axis_index("s")
        w = sid * 2 + c
        pltpu.sync_copy(srcr_hbm.at[w], sidx)
        pltpu.sync_copy(dstr_hbm.at[w], didx)
        pltpu.sync_copy(zeros_hbm.at[pl.ds(sid * ZR, ZR)],
                        acc_sh.at[pl.ds(sid * ZR, ZR)])
        plsc.subcore_barrier()

        @pl.loop(0, NCHL)
        def _(j):
            pltpu.async_copy(h_hbm.at[sidx.at[j]], rows_v, gsem).wait()
            pltpu.sync_copy(rows_v, acc_sh.at[didx.at[j]], add=True)

        plsc.subcore_barrier()
        pltpu.sync_copy(acc_sh.at[pl.ds(sid * ZR, ZR)],
                        out_hbm.at[c, pl.ds(sid * ZR, ZR)])

    return k


def _tc_layer(p, h_prev, A, Br, bias):
    Dp = p.shape[2]
    Din = h_prev.shape[1]
    BRW = 512

    def body(p_ref, h_ref, a_ref, b_ref, bias_ref, o_ref):
        agg = p_ref[0] + p_ref[1]
        acc = jnp.dot(agg, a_ref[...], preferred_element_type=jnp.float32)
        acc = acc + jnp.dot(h_ref[...], b_ref[...],
                            preferred_element_type=jnp.float32)
        acc = acc + bias_ref[...]
        i = pl.program_id(0)
        rows = i * BRW + lax.broadcasted_iota(jnp.int32, (BRW, 1), 0)
        o_ref[...] = jnp.where(rows < N, jnp.maximum(acc, 0.0), 0.0)

    return pl.pallas_call(
        body,
        grid=(NP // BRW,),
        in_specs=[
            pl.BlockSpec((2, BRW, Dp), lambda i: (0, i, 0)),
            pl.BlockSpec((BRW, Din), lambda i: (i, 0)),
            pl.BlockSpec((Dp, H), lambda i: (0, 0)),
            pl.BlockSpec((Din, H), lambda i: (0, 0)),
            pl.BlockSpec((1, H), lambda i: (0, 0)),
        ],
        out_specs=pl.BlockSpec((BRW, H), lambda i: (i, 0)),
        out_shape=jax.ShapeDtypeStruct((NP, H), jnp.float32),
    )(p, h_prev, A, Br, bias)


BG = 7


def _tc_pool(xr, adj_p, s_p):
    def body(xr_ref, adj_ref, s_ref, out_ref, oadj_ref, acc_ref):
        g = pl.program_id(0)

        @pl.when(g == 0)
        def _():
            acc_ref[0, 0] = 0.0
            acc_ref[0, 1] = 0.0

        link_tot = jnp.float32(0.0)
        ent_tot = jnp.float32(0.0)
        rows = lax.broadcasted_iota(jnp.int32, (128, 1), 0)
        for t in range(BG):
            sg = s_ref[t]
            m = jnp.exp(sg - jnp.max(sg, axis=-1, keepdims=True))
            ssm = m / jnp.sum(m, axis=-1, keepdims=True)
            ssm = jnp.where(rows < NN, ssm, 0.0)
            xg = xr_ref[t]
            ag = adj_ref[t]
            out_ref[t] = lax.dot_general(
                ssm, xg, (((0,), (0,)), ((), ())),
                preferred_element_type=jnp.float32)
            ta = lax.dot_general(
                ssm, ag, (((0,), (0,)), ((), ())),
                preferred_element_type=jnp.float32)
            oadj_ref[t] = lax.dot_general(
                ta, ssm, (((1,), (0,)), ((), ())),
                preferred_element_type=jnp.float32)
            link = ag - lax.dot_general(
                ssm, ssm, (((1,), (1,)), ((), ())),
                preferred_element_type=jnp.float32)
            link_tot = link_tot + jnp.sum(link * link)
            ent_tot = ent_tot + jnp.sum(-ssm * jnp.log(ssm + EPS))
        acc_ref[0, 0] += link_tot
        acc_ref[0, 1] += ent_tot

    return pl.pallas_call(
        body,
        grid=(NG // BG,),
        in_specs=[
            pl.BlockSpec((BG, 128, H), lambda g: (g, 0, 0)),
            pl.BlockSpec((BG, 128, 128), lambda g: (g, 0, 0)),
            pl.BlockSpec((BG, 128, C), lambda g: (g, 0, 0)),
        ],
        out_specs=[
            pl.BlockSpec((BG, C, H), lambda g: (g, 0, 0)),
            pl.BlockSpec((BG, C, C), lambda g: (g, 0, 0)),
            pl.BlockSpec(memory_space=pltpu.SMEM),
        ],
        out_shape=[
            jax.ShapeDtypeStruct((NG, C, H), jnp.float32),
            jax.ShapeDtypeStruct((NG, C, C), jnp.float32),
            jax.ShapeDtypeStruct((1, 2), jnp.float32),
        ],
    )(xr, adj_p, s_p)


def kernel(x, edge_index, adj, s, pos,
           W_rel1, b_rel1, W_root1,
           W_rel2, b_rel2, W_root2,
           W_rel3, b_rel3, W_root3):
    src = edge_index[0]
    dst = edge_index[1]
    padn = NW * NCH * K - E
    dummy = jnp.full((NW, CHT - NCH, K), PADROW, jnp.int32)

    def _reshape_idx(v):
        vr = jnp.concatenate(
            [v, jnp.full((padn,), PADROW, jnp.int32)]).reshape(NW, NCH, K)
        return jnp.concatenate([vr, dummy], axis=1)

    srcr = _reshape_idx(src)
    dstr = _reshape_idx(dst)

    x16 = jnp.zeros((NP, 16), jnp.float32).at[:N, :2].set(x)
    z16 = jnp.zeros((NP, 16), jnp.float32)
    z128 = jnp.zeros((NP, H), jnp.float32)

    A1 = jnp.zeros((16, H), jnp.float32).at[:2, :].set(W_rel1.T)
    B1 = jnp.zeros((16, H), jnp.float32).at[:2, :].set(W_root1.T)

    seg16 = _sc_segsum(16)
    seg128 = _sc_segsum(H)

    p1 = seg16(x16, srcr, dstr, z16)
    h1 = _tc_layer(p1, x16, A1, B1, b_rel1.reshape(1, H))

    p2 = seg128(h1, srcr, dstr, z128)
    h2 = _tc_layer(p2, h1, W_rel2.T, W_root2.T, b_rel2.reshape(1, H))

    p3 = seg128(h2, srcr, dstr, z128)
    h3 = _tc_layer(p3, h2, W_rel3.T, W_root3.T, b_rel3.reshape(1, H))

    xr = h3[:N].reshape(NG, NN, H)
    xr_p = jnp.zeros((NG, 128, H), jnp.float32).at[:, :NN, :].set(xr)
    adj_p = jnp.zeros((NG, 128, 128), jnp.float32).at[:, :NN, :NN].set(adj)
    s_p = jnp.zeros((NG, 128, C), jnp.float32).at[:, :NN, :].set(s)

    out, out_adj, acc = _tc_pool(xr_p, adj_p, s_p)
    link_loss = jnp.sqrt(acc[0, 0]) / (NG * NN * NN)
    ent_loss = acc[0, 1] / (NG * NN)
    return out, out_adj, link_loss, ent_loss, pos


# speedup vs baseline: 1.6021x; 1.1286x over previous
import functools

import jax
import jax.numpy as jnp
from jax import lax
from jax.experimental import pallas as pl
from jax.experimental.pallas import tpu as pltpu
from jax.experimental.pallas import tpu_sc as plsc

N = 9996
E = 319872
NG = 147
NN = 68
H = 128
C = 16
EPS = 1e-15

NP = 10240
PADROW = NP - 1
NW = 32
K = 128
EW = (E + NW * K - 1) // (NW * K) * K
NCH = EW // K
NCHL = NCH
CHT = NCH
ZR = NP // 16


def _sc_segsum(D):
    mesh = plsc.VectorSubcoreMesh(core_axis_name="c", subcore_axis_name="s")

    @functools.partial(
        pl.kernel,
        out_type=jax.ShapeDtypeStruct((2, NP, D), jnp.float32),
        mesh=mesh,
        compiler_params=pltpu.CompilerParams(use_tc_tiling_on_sc=False),
        scratch_types=[
            pltpu.VMEM((CHT, K), jnp.int32),
            pltpu.VMEM((CHT, K), jnp.int32),
            pltpu.VMEM((K, D), jnp.float32),
            pltpu.VMEM_SHARED((NP, D), jnp.float32),
            pltpu.SemaphoreType.DMA,
        ],
    )
    def k(h_hbm, srcr_hbm, dstr_hbm, zeros_hbm, out_hbm,
          sidx, didx, rows_v, acc_sh, gsem):
        c = lax.axis_index("c")
        sid = lax.axis_index("s")
        w = sid * 2 + c
        pltpu.sync_copy(srcr_hbm.at[w], sidx)
        pltpu.sync_copy(dstr_hbm.at[w], didx)
        pltpu.sync_copy(zeros_hbm.at[pl.ds(sid * ZR, ZR)],
                        acc_sh.at[pl.ds(sid * ZR, ZR)])
        plsc.subcore_barrier()

        @pl.loop(0, NCHL)
        def _(j):
            pltpu.async_copy(h_hbm.at[sidx.at[j]], rows_v, gsem).wait()

        plsc.subcore_barrier()
        pltpu.sync_copy(acc_sh.at[pl.ds(sid * ZR, ZR)],
                        out_hbm.at[c, pl.ds(sid * ZR, ZR)])

    return k


def _tc_layer(p, h_prev, A, Br, bias):
    Dp = p.shape[2]
    Din = h_prev.shape[1]
    BRW = 512

    def body(p_ref, h_ref, a_ref, b_ref, bias_ref, o_ref):
        agg = p_ref[0] + p_ref[1]
        acc = jnp.dot(agg, a_ref[...], preferred_element_type=jnp.float32)
        acc = acc + jnp.dot(h_ref[...], b_ref[...],
                            preferred_element_type=jnp.float32)
        acc = acc + bias_ref[...]
        i = pl.program_id(0)
        rows = i * BRW + lax.broadcasted_iota(jnp.int32, (BRW, 1), 0)
        o_ref[...] = jnp.where(rows < N, jnp.maximum(acc, 0.0), 0.0)

    return pl.pallas_call(
        body,
        grid=(NP // BRW,),
        in_specs=[
            pl.BlockSpec((2, BRW, Dp), lambda i: (0, i, 0)),
            pl.BlockSpec((BRW, Din), lambda i: (i, 0)),
            pl.BlockSpec((Dp, H), lambda i: (0, 0)),
            pl.BlockSpec((Din, H), lambda i: (0, 0)),
            pl.BlockSpec((1, H), lambda i: (0, 0)),
        ],
        out_specs=pl.BlockSpec((BRW, H), lambda i: (i, 0)),
        out_shape=jax.ShapeDtypeStruct((NP, H), jnp.float32),
    )(p, h_prev, A, Br, bias)


BG = 7


def _tc_pool(xr, adj_p, s_p):
    def body(xr_ref, adj_ref, s_ref, out_ref, oadj_ref, acc_ref):
        g = pl.program_id(0)

        @pl.when(g == 0)
        def _():
            acc_ref[0, 0] = 0.0
            acc_ref[0, 1] = 0.0

        link_tot = jnp.float32(0.0)
        ent_tot = jnp.float32(0.0)
        rows = lax.broadcasted_iota(jnp.int32, (128, 1), 0)
        for t in range(BG):
            sg = s_ref[t]
            m = jnp.exp(sg - jnp.max(sg, axis=-1, keepdims=True))
            ssm = m / jnp.sum(m, axis=-1, keepdims=True)
            ssm = jnp.where(rows < NN, ssm, 0.0)
            xg = xr_ref[t]
            ag = adj_ref[t]
            out_ref[t] = lax.dot_general(
                ssm, xg, (((0,), (0,)), ((), ())),
                preferred_element_type=jnp.float32)
            ta = lax.dot_general(
                ssm, ag, (((0,), (0,)), ((), ())),
                preferred_element_type=jnp.float32)
            oadj_ref[t] = lax.dot_general(
                ta, ssm, (((1,), (0,)), ((), ())),
                preferred_element_type=jnp.float32)
            link = ag - lax.dot_general(
                ssm, ssm, (((1,), (1,)), ((), ())),
                preferred_element_type=jnp.float32)
            link_tot = link_tot + jnp.sum(link * link)
            ent_tot = ent_tot + jnp.sum(-ssm * jnp.log(ssm + EPS))
        acc_ref[0, 0] += link_tot
        acc_ref[0, 1] += ent_tot

    return pl.pallas_call(
        body,
        grid=(NG // BG,),
        in_specs=[
            pl.BlockSpec((BG, 128, H), lambda g: (g, 0, 0)),
            pl.BlockSpec((BG, 128, 128), lambda g: (g, 0, 0)),
            pl.BlockSpec((BG, 128, C), lambda g: (g, 0, 0)),
        ],
        out_specs=[
            pl.BlockSpec((BG, C, H), lambda g: (g, 0, 0)),
            pl.BlockSpec((BG, C, C), lambda g: (g, 0, 0)),
            pl.BlockSpec(memory_space=pltpu.SMEM),
        ],
        out_shape=[
            jax.ShapeDtypeStruct((NG, C, H), jnp.float32),
            jax.ShapeDtypeStruct((NG, C, C), jnp.float32),
            jax.ShapeDtypeStruct((1, 2), jnp.float32),
        ],
    )(xr, adj_p, s_p)


def kernel(x, edge_index, adj, s, pos,
           W_rel1, b_rel1, W_root1,
           W_rel2, b_rel2, W_root2,
           W_rel3, b_rel3, W_root3):
    src = edge_index[0]
    dst = edge_index[1]
    padn = NW * NCH * K - E
    dummy = jnp.full((NW, CHT - NCH, K), PADROW, jnp.int32)

    def _reshape_idx(v):
        vr = jnp.concatenate(
            [v, jnp.full((padn,), PADROW, jnp.int32)]).reshape(NW, NCH, K)
        return jnp.concatenate([vr, dummy], axis=1)

    srcr = _reshape_idx(src)
    dstr = _reshape_idx(dst)

    x16 = jnp.zeros((NP, 16), jnp.float32).at[:N, :2].set(x)
    z16 = jnp.zeros((NP, 16), jnp.float32)
    z128 = jnp.zeros((NP, H), jnp.float32)

    A1 = jnp.zeros((16, H), jnp.float32).at[:2, :].set(W_rel1.T)
    B1 = jnp.zeros((16, H), jnp.float32).at[:2, :].set(W_root1.T)

    seg16 = _sc_segsum(16)
    seg128 = _sc_segsum(H)

    p1 = seg16(x16, srcr, dstr, z16)
    h1 = _tc_layer(p1, x16, A1, B1, b_rel1.reshape(1, H))

    p2 = seg128(h1, srcr, dstr, z128)
    h2 = _tc_layer(p2, h1, W_rel2.T, W_root2.T, b_rel2.reshape(1, H))

    p3 = seg128(h2, srcr, dstr, z128)
    h3 = _tc_layer(p3, h2, W_rel3.T, W_root3.T, b_rel3.reshape(1, H))

    xr = h3[:N].reshape(NG, NN, H)
    xr_p = jnp.zeros((NG, 128, H), jnp.float32).at[:, :NN, :].set(xr)
    adj_p = jnp.zeros((NG, 128, 128), jnp.float32).at[:, :NN, :NN].set(adj)
    s_p = jnp.zeros((NG, 128, C), jnp.float32).at[:, :NN, :].set(s)

    out, out_adj, acc = _tc_pool(xr_p, adj_p, s_p)
    link_loss = jnp.sqrt(acc[0, 0]) / (NG * NN * NN)
    ent_loss = acc[0, 1] / (NG * NN)
    return out, out_adj, link_loss, ent_loss, pos


# speedup vs baseline: 1.8082x; 1.1286x over previous
import functools

import jax
import jax.numpy as jnp
from jax import lax
from jax.experimental import pallas as pl
from jax.experimental.pallas import tpu as pltpu
from jax.experimental.pallas import tpu_sc as plsc

N = 9996
E = 319872
NG = 147
NN = 68
H = 128
C = 16
EPS = 1e-15

NP = 10240
PADROW = NP - 1
NW = 32
K = 128
EW = (E + NW * K - 1) // (NW * K) * K
NCH = EW // K
ZR = NP // 16

DH = 64
NW2 = 16
EW2 = (E + NW2 * K - 1) // (NW2 * K) * K
NCH2 = EW2 // K


def _sc_segsum16(D):
    mesh = plsc.VectorSubcoreMesh(core_axis_name="c", subcore_axis_name="s")

    @functools.partial(
        pl.kernel,
        out_type=jax.ShapeDtypeStruct((2, NP, D), jnp.float32),
        mesh=mesh,
        compiler_params=pltpu.CompilerParams(use_tc_tiling_on_sc=False),
        scratch_types=[
            pltpu.VMEM((NCH, K), jnp.int32),
            pltpu.VMEM((NCH, K), jnp.int32),
            pltpu.VMEM((K, D), jnp.float32),
            pltpu.VMEM_SHARED((NP, D), jnp.float32),
            pltpu.SemaphoreType.DMA,
        ],
    )
    def k(h_hbm, srcr_hbm, dstr_hbm, zeros_hbm, out_hbm,
          sidx, didx, rows_v, acc_sh, gsem):
        c = lax.axis_index("c")
        sid = lax.axis_index("s")
        w = sid * 2 + c
        pltpu.sync_copy(srcr_hbm.at[w], sidx)
        pltpu.sync_copy(dstr_hbm.at[w], didx)
        pltpu.sync_copy(zeros_hbm.at[pl.ds(sid * ZR, ZR)],
                        acc_sh.at[pl.ds(sid * ZR, ZR)])
        plsc.subcore_barrier()

        @pl.loop(0, NCH)
        def _(j):
            pltpu.async_copy(h_hbm.at[sidx.at[j]], rows_v, gsem).wait()
            pltpu.sync_copy(rows_v, acc_sh.at[didx.at[j]], add=True)

        plsc.subcore_barrier()
        pltpu.sync_copy(acc_sh.at[pl.ds(sid * ZR, ZR)],
                        out_hbm.at[c, pl.ds(sid * ZR, ZR)])

    return k


def _sc_segsum_colsplit():
    mesh = plsc.VectorSubcoreMesh(core_axis_name="c", subcore_axis_name="s")

    @functools.partial(
        pl.kernel,
        out_type=jax.ShapeDtypeStruct((2, NP, DH), jnp.float32),
        mesh=mesh,
        compiler_params=pltpu.CompilerParams(use_tc_tiling_on_sc=False),
        scratch_types=[
            pltpu.VMEM((NCH2, K), jnp.int32),
            pltpu.VMEM((NCH2, K), jnp.int32),
            pltpu.VMEM((K, DH), jnp.float32),
            pltpu.VMEM_SHARED((NP, DH), jnp.float32),
            pltpu.VMEM_SHARED((NP, DH), jnp.float32),
            pltpu.SemaphoreType.DMA,
        ],
    )
    def k(h2_hbm, srcr_hbm, dstr_hbm, zeros_hbm, out_hbm,
          sidx, didx, rows_v, h_sh, acc_sh, gsem):
        c = lax.axis_index("c")
        sid = lax.axis_index("s")
        pltpu.sync_copy(srcr_hbm.at[sid], sidx)
        pltpu.sync_copy(dstr_hbm.at[sid], didx)
        pltpu.sync_copy(zeros_hbm.at[pl.ds(sid * ZR, ZR)],
                        acc_sh.at[pl.ds(sid * ZR, ZR)])
        pltpu.sync_copy(h2_hbm.at[c, pl.ds(sid * ZR, ZR)],
                        h_sh.at[pl.ds(sid * ZR, ZR)])
        plsc.subcore_barrier()

        @pl.loop(0, NCH2)
        def _(j):
            pltpu.async_copy(h_sh.at[sidx.at[j]], rows_v, gsem).wait()
            pltpu.sync_copy(rows_v, acc_sh.at[didx.at[j]], add=True)

        plsc.subcore_barrier()
        pltpu.sync_copy(acc_sh.at[pl.ds(sid * ZR, ZR)],
                        out_hbm.at[c, pl.ds(sid * ZR, ZR)])

    return k


BRW = 512


def _tc_layer1(p, x16, A, Br, bias):
    def body(p_ref, h_ref, a_ref, b_ref, bias_ref, o_ref):
        agg = p_ref[0] + p_ref[1]
        acc = jnp.dot(agg, a_ref[...], preferred_element_type=jnp.float32)
        acc = acc + jnp.dot(h_ref[...], b_ref[...],
                            preferred_element_type=jnp.float32)
        acc = acc + bias_ref[...]
        i = pl.program_id(0)
        rows = i * BRW + lax.broadcasted_iota(jnp.int32, (BRW, 1), 0)
        res = jnp.where(rows < N, jnp.maximum(acc, 0.0), 0.0)
        o_ref[0] = res[:, :DH]
        o_ref[1] = res[:, DH:]

    return pl.pallas_call(
        body,
        grid=(NP // BRW,),
        in_specs=[
            pl.BlockSpec((2, BRW, 16), lambda i: (0, i, 0)),
            pl.BlockSpec((BRW, 16), lambda i: (i, 0)),
            pl.BlockSpec((16, H), lambda i: (0, 0)),
            pl.BlockSpec((16, H), lambda i: (0, 0)),
            pl.BlockSpec((1, H), lambda i: (0, 0)),
        ],
        out_specs=pl.BlockSpec((2, BRW, DH), lambda i: (0, i, 0)),
        out_shape=jax.ShapeDtypeStruct((2, NP, DH), jnp.float32),
    )(p, x16, A, Br, bias)


def _tc_layer(p, h_prev, A, Br, bias, split_out):
    def body(p_ref, h_ref, a_ref, b_ref, bias_ref, o_ref):
        agg = jnp.concatenate([p_ref[0], p_ref[1]], axis=1)
        hp = jnp.concatenate([h_ref[0], h_ref[1]], axis=1)
        acc = jnp.dot(agg, a_ref[...], preferred_element_type=jnp.float32)
        acc = acc + jnp.dot(hp, b_ref[...],
                            preferred_element_type=jnp.float32)
        acc = acc + bias_ref[...]
        i = pl.program_id(0)
        rows = i * BRW + lax.broadcasted_iota(jnp.int32, (BRW, 1), 0)
        res = jnp.where(rows < N, jnp.maximum(acc, 0.0), 0.0)
        if split_out:
            o_ref[0] = res[:, :DH]
            o_ref[1] = res[:, DH:]
        else:
            o_ref[...] = res

    if split_out:
        out_spec = pl.BlockSpec((2, BRW, DH), lambda i: (0, i, 0))
        out_shape = jax.ShapeDtypeStruct((2, NP, DH), jnp.float32)
    else:
        out_spec = pl.BlockSpec((BRW, H), lambda i: (i, 0))
        out_shape = jax.ShapeDtypeStruct((NP, H), jnp.float32)

    return pl.pallas_call(
        body,
        grid=(NP // BRW,),
        in_specs=[
            pl.BlockSpec((2, BRW, DH), lambda i: (0, i, 0)),
            pl.BlockSpec((2, BRW, DH), lambda i: (0, i, 0)),
            pl.BlockSpec((H, H), lambda i: (0, 0)),
            pl.BlockSpec((H, H), lambda i: (0, 0)),
            pl.BlockSpec((1, H), lambda i: (0, 0)),
        ],
        out_specs=out_spec,
        out_shape=out_shape,
    )(p, h_prev, A, Br, bias)


BG = 7


def _tc_pool(xr, adj_p, s_p):
    def body(xr_ref, adj_ref, s_ref, out_ref, oadj_ref, acc_ref):
        g = pl.program_id(0)

        @pl.when(g == 0)
        def _():
            acc_ref[0, 0] = 0.0
            acc_ref[0, 1] = 0.0

        link_tot = jnp.float32(0.0)
        ent_tot = jnp.float32(0.0)
        rows = lax.broadcasted_iota(jnp.int32, (128, 1), 0)
        for t in range(BG):
            sg = s_ref[t]
            m = jnp.exp(sg - jnp.max(sg, axis=-1, keepdims=True))
            ssm = m / jnp.sum(m, axis=-1, keepdims=True)
            ssm = jnp.where(rows < NN, ssm, 0.0)
            xg = xr_ref[t]
            ag = adj_ref[t]
            out_ref[t] = lax.dot_general(
                ssm, xg, (((0,), (0,)), ((), ())),
                preferred_element_type=jnp.float32)
            ta = lax.dot_general(
                ssm, ag, (((0,), (0,)), ((), ())),
                preferred_element_type=jnp.float32)
            oadj_ref[t] = lax.dot_general(
                ta, ssm, (((1,), (0,)), ((), ())),
                preferred_element_type=jnp.float32)
            link = ag - lax.dot_general(
                ssm, ssm, (((1,), (1,)), ((), ())),
                preferred_element_type=jnp.float32)
            link_tot = link_tot + jnp.sum(link * link)
            ent_tot = ent_tot + jnp.sum(-ssm * jnp.log(ssm + EPS))
        acc_ref[0, 0] += link_tot
        acc_ref[0, 1] += ent_tot

    return pl.pallas_call(
        body,
        grid=(NG // BG,),
        in_specs=[
            pl.BlockSpec((BG, 128, H), lambda g: (g, 0, 0)),
            pl.BlockSpec((BG, 128, 128), lambda g: (g, 0, 0)),
            pl.BlockSpec((BG, 128, C), lambda g: (g, 0, 0)),
        ],
        out_specs=[
            pl.BlockSpec((BG, C, H), lambda g: (g, 0, 0)),
            pl.BlockSpec((BG, C, C), lambda g: (g, 0, 0)),
            pl.BlockSpec(memory_space=pltpu.SMEM),
        ],
        out_shape=[
            jax.ShapeDtypeStruct((NG, C, H), jnp.float32),
            jax.ShapeDtypeStruct((NG, C, C), jnp.float32),
            jax.ShapeDtypeStruct((1, 2), jnp.float32),
        ],
    )(xr, adj_p, s_p)


def _reshape_idx(v, nw, nch):
    padn = nw * nch * K - E
    return jnp.concatenate(
        [v, jnp.full((padn,), PADROW, jnp.int32)]).reshape(nw, nch, K)


def kernel(x, edge_index, adj, s, pos,
           W_rel1, b_rel1, W_root1,
           W_rel2, b_rel2, W_root2,
           W_rel3, b_rel3, W_root3):
    src = edge_index[0]
    dst = edge_index[1]

    srcr1 = _reshape_idx(src, NW, NCH)
    dstr1 = _reshape_idx(dst, NW, NCH)
    srcr2 = _reshape_idx(src, NW2, NCH2)
    dstr2 = _reshape_idx(dst, NW2, NCH2)

    x16 = jnp.zeros((NP, 16), jnp.float32).at[:N, :2].set(x)
    z16 = jnp.zeros((NP, 16), jnp.float32)
    z64 = jnp.zeros((NP, DH), jnp.float32)

    A1 = jnp.zeros((16, H), jnp.float32).at[:2, :].set(W_rel1.T)
    B1 = jnp.zeros((16, H), jnp.float32).at[:2, :].set(W_root1.T)

    seg16 = _sc_segsum16(16)
    seg64 = _sc_segsum_colsplit()

    p1 = seg16(x16, srcr1, dstr1, z16)
    h1 = _tc_layer1(p1, x16, A1, B1, b_rel1.reshape(1, H))

    p2 = seg64(h1, srcr2, dstr2, z64)
    h2 = _tc_layer(p2, h1, W_rel2.T, W_root2.T, b_rel2.reshape(1, H),
                   split_out=True)

    p3 = seg64(h2, srcr2, dstr2, z64)
    h3 = _tc_layer(p3, h2, W_rel3.T, W_root3.T, b_rel3.reshape(1, H),
                   split_out=False)

    xr = h3[:N].reshape(NG, NN, H)
    xr_p = jnp.zeros((NG, 128, H), jnp.float32).at[:, :NN, :].set(xr)
    adj_p = jnp.zeros((NG, 128, 128), jnp.float32).at[:, :NN, :NN].set(adj)
    s_p = jnp.zeros((NG, 128, C), jnp.float32).at[:, :NN, :].set(s)

    out, out_adj, acc = _tc_pool(xr_p, adj_p, s_p)
    link_loss = jnp.sqrt(acc[0, 0]) / (NG * NN * NN)
    ent_loss = acc[0, 1] / (NG * NN)
    return out, out_adj, link_loss, ent_loss, pos


# speedup vs baseline: 2.1447x; 1.1861x over previous
import functools

import jax
import jax.numpy as jnp
from jax import lax
from jax.experimental import pallas as pl
from jax.experimental.pallas import tpu as pltpu
from jax.experimental.pallas import tpu_sc as plsc

N = 9996
E = 319872
NG = 147
NN = 68
H = 128
C = 16
EPS = 1e-15

NP = 10240
PADROW = NP - 1
NW = 32
K = 128
EW = (E + NW * K - 1) // (NW * K) * K
NCH = EW // K
ZR = NP // 16

DH = 64
NW2 = 16
NPASS = 2
EW2 = (E + NW2 * NPASS * K - 1) // (NW2 * NPASS * K) * K
NCH2H = EW2 // K
NCH2 = NPASS * NCH2H


def _sc_segsum16(D):
    mesh = plsc.VectorSubcoreMesh(core_axis_name="c", subcore_axis_name="s")

    @functools.partial(
        pl.kernel,
        out_type=jax.ShapeDtypeStruct((2, NP, D), jnp.float32),
        mesh=mesh,
        compiler_params=pltpu.CompilerParams(use_tc_tiling_on_sc=False),
        scratch_types=[
            pltpu.VMEM((NCH, K), jnp.int32),
            pltpu.VMEM((NCH, K), jnp.int32),
            pltpu.VMEM((K, D), jnp.float32),
            pltpu.VMEM_SHARED((NP, D), jnp.float32),
            pltpu.SemaphoreType.DMA,
        ],
    )
    def k(h_hbm, srcr_hbm, dstr_hbm, zeros_hbm, out_hbm,
          sidx, didx, rows_v, acc_sh, gsem):
        c = lax.axis_index("c")
        sid = lax.axis_index("s")
        w = sid * 2 + c
        pltpu.sync_copy(srcr_hbm.at[w], sidx)
        pltpu.sync_copy(dstr_hbm.at[w], didx)
        pltpu.sync_copy(zeros_hbm.at[pl.ds(sid * ZR, ZR)],
                        acc_sh.at[pl.ds(sid * ZR, ZR)])
        plsc.subcore_barrier()

        @pl.loop(0, NCH)
        def _(j):
            pltpu.async_copy(h_hbm.at[sidx.at[j]], rows_v, gsem).wait()
            pltpu.sync_copy(rows_v, acc_sh.at[didx.at[j]], add=True)

        plsc.subcore_barrier()
        pltpu.sync_copy(acc_sh.at[pl.ds(sid * ZR, ZR)],
                        out_hbm.at[c, pl.ds(sid * ZR, ZR)])

    return k


def _sc_segsum_colsplit():
    mesh = plsc.VectorSubcoreMesh(core_axis_name="c", subcore_axis_name="s")

    @functools.partial(
        pl.kernel,
        out_type=jax.ShapeDtypeStruct((2, NP, DH), jnp.float32),
        mesh=mesh,
        compiler_params=pltpu.CompilerParams(use_tc_tiling_on_sc=False),
        scratch_types=[
            pltpu.VMEM((NCH2H, K), jnp.int32),
            pltpu.VMEM((NCH2H, K), jnp.int32),
            pltpu.VMEM((2, K, DH), jnp.float32),
            pltpu.VMEM_SHARED((NP, DH), jnp.float32),
            pltpu.VMEM_SHARED((NP, DH), jnp.float32),
            pltpu.SemaphoreType.DMA,
        ],
    )
    def k(h2_hbm, srcr_hbm, dstr_hbm, zeros_hbm, out_hbm,
          sidx, didx, rows_v, h_sh, acc_sh, gsem):
        c = lax.axis_index("c")
        sid = lax.axis_index("s")
        pltpu.sync_copy(zeros_hbm.at[pl.ds(sid * ZR, ZR)],
                        acc_sh.at[pl.ds(sid * ZR, ZR)])
        pltpu.sync_copy(h2_hbm.at[c, pl.ds(sid * ZR, ZR)],
                        h_sh.at[pl.ds(sid * ZR, ZR)])
        plsc.subcore_barrier()

        for p in range(NPASS):
            t = sid * NPASS + p
            pltpu.sync_copy(srcr_hbm.at[t], sidx)
            pltpu.sync_copy(dstr_hbm.at[t], didx)
            pltpu.async_copy(h_sh.at[sidx.at[0]], rows_v.at[0], gsem)

            @pl.loop(0, NCH2H - 1)
            def _(j):
                pltpu.make_async_copy(h_sh.at[sidx.at[j]],
                                      rows_v.at[j % 2], gsem).wait()
                pltpu.async_copy(h_sh.at[sidx.at[j + 1]],
                                 rows_v.at[(j + 1) % 2], gsem)
                pltpu.sync_copy(rows_v.at[j % 2], acc_sh.at[didx.at[j]],
                                add=True)

            pltpu.make_async_copy(h_sh.at[sidx.at[NCH2H - 1]],
                                  rows_v.at[(NCH2H - 1) % 2], gsem).wait()
            pltpu.sync_copy(rows_v.at[(NCH2H - 1) % 2],
                            acc_sh.at[didx.at[NCH2H - 1]], add=True)

        plsc.subcore_barrier()
        pltpu.sync_copy(acc_sh.at[pl.ds(sid * ZR, ZR)],
                        out_hbm.at[c, pl.ds(sid * ZR, ZR)])

    return k


BRW = 512


def _tc_layer1(p, x16, A, Br, bias):
    def body(p_ref, h_ref, a_ref, b_ref, bias_ref, o_ref):
        agg = p_ref[0] + p_ref[1]
        acc = jnp.dot(agg, a_ref[...], preferred_element_type=jnp.float32)
        acc = acc + jnp.dot(h_ref[...], b_ref[...],
                            preferred_element_type=jnp.float32)
        acc = acc + bias_ref[...]
        i = pl.program_id(0)
        rows = i * BRW + lax.broadcasted_iota(jnp.int32, (BRW, 1), 0)
        res = jnp.where(rows < N, jnp.maximum(acc, 0.0), 0.0)
        o_ref[0] = res[:, :DH]
        o_ref[1] = res[:, DH:]

    return pl.pallas_call(
        body,
        grid=(NP // BRW,),
        in_specs=[
            pl.BlockSpec((2, BRW, 16), lambda i: (0, i, 0)),
            pl.BlockSpec((BRW, 16), lambda i: (i, 0)),
            pl.BlockSpec((16, H), lambda i: (0, 0)),
            pl.BlockSpec((16, H), lambda i: (0, 0)),
            pl.BlockSpec((1, H), lambda i: (0, 0)),
        ],
        out_specs=pl.BlockSpec((2, BRW, DH), lambda i: (0, i, 0)),
        out_shape=jax.ShapeDtypeStruct((2, NP, DH), jnp.float32),
    )(p, x16, A, Br, bias)


def _tc_layer(p, h_prev, A, Br, bias, split_out):
    def body(p_ref, h_ref, a_ref, b_ref, bias_ref, o_ref):
        agg = jnp.concatenate([p_ref[0], p_ref[1]], axis=1)
        hp = jnp.concatenate([h_ref[0], h_ref[1]], axis=1)
        acc = jnp.dot(agg, a_ref[...], preferred_element_type=jnp.float32)
        acc = acc + jnp.dot(hp, b_ref[...],
                            preferred_element_type=jnp.float32)
        acc = acc + bias_ref[...]
        i = pl.program_id(0)
        rows = i * BRW + lax.broadcasted_iota(jnp.int32, (BRW, 1), 0)
        res = jnp.where(rows < N, jnp.maximum(acc, 0.0), 0.0)
        if split_out:
            o_ref[0] = res[:, :DH]
            o_ref[1] = res[:, DH:]
        else:
            o_ref[...] = res

    if split_out:
        out_spec = pl.BlockSpec((2, BRW, DH), lambda i: (0, i, 0))
        out_shape = jax.ShapeDtypeStruct((2, NP, DH), jnp.float32)
    else:
        out_spec = pl.BlockSpec((BRW, H), lambda i: (i, 0))
        out_shape = jax.ShapeDtypeStruct((NP, H), jnp.float32)

    return pl.pallas_call(
        body,
        grid=(NP // BRW,),
        in_specs=[
            pl.BlockSpec((2, BRW, DH), lambda i: (0, i, 0)),
            pl.BlockSpec((2, BRW, DH), lambda i: (0, i, 0)),
            pl.BlockSpec((H, H), lambda i: (0, 0)),
            pl.BlockSpec((H, H), lambda i: (0, 0)),
            pl.BlockSpec((1, H), lambda i: (0, 0)),
        ],
        out_specs=out_spec,
        out_shape=out_shape,
    )(p, h_prev, A, Br, bias)


BG = 7


def _tc_pool(xr, adj_p, s_p):
    def body(xr_ref, adj_ref, s_ref, out_ref, oadj_ref, acc_ref):
        g = pl.program_id(0)

        @pl.when(g == 0)
        def _():
            acc_ref[0, 0] = 0.0
            acc_ref[0, 1] = 0.0

        link_tot = jnp.float32(0.0)
        ent_tot = jnp.float32(0.0)
        rows = lax.broadcasted_iota(jnp.int32, (128, 1), 0)
        for t in range(BG):
            sg = s_ref[t]
            m = jnp.exp(sg - jnp.max(sg, axis=-1, keepdims=True))
            ssm = m / jnp.sum(m, axis=-1, keepdims=True)
            ssm = jnp.where(rows < NN, ssm, 0.0)
            xg = xr_ref[t]
            ag = adj_ref[t]
            out_ref[t] = lax.dot_general(
                ssm, xg, (((0,), (0,)), ((), ())),
                preferred_element_type=jnp.float32)
            ta = lax.dot_general(
                ssm, ag, (((0,), (0,)), ((), ())),
                preferred_element_type=jnp.float32)
            oadj_ref[t] = lax.dot_general(
                ta, ssm, (((1,), (0,)), ((), ())),
                preferred_element_type=jnp.float32)
            link = ag - lax.dot_general(
                ssm, ssm, (((1,), (1,)), ((), ())),
                preferred_element_type=jnp.float32)
            link_tot = link_tot + jnp.sum(link * link)
            ent_tot = ent_tot + jnp.sum(-ssm * jnp.log(ssm + EPS))
        acc_ref[0, 0] += link_tot
        acc_ref[0, 1] += ent_tot

    return pl.pallas_call(
        body,
        grid=(NG // BG,),
        in_specs=[
            pl.BlockSpec((BG, 128, H), lambda g: (g, 0, 0)),
            pl.BlockSpec((BG, 128, 128), lambda g: (g, 0, 0)),
            pl.BlockSpec((BG, 128, C), lambda g: (g, 0, 0)),
        ],
        out_specs=[
            pl.BlockSpec((BG, C, H), lambda g: (g, 0, 0)),
            pl.BlockSpec((BG, C, C), lambda g: (g, 0, 0)),
            pl.BlockSpec(memory_space=pltpu.SMEM),
        ],
        out_shape=[
            jax.ShapeDtypeStruct((NG, C, H), jnp.float32),
            jax.ShapeDtypeStruct((NG, C, C), jnp.float32),
            jax.ShapeDtypeStruct((1, 2), jnp.float32),
        ],
    )(xr, adj_p, s_p)


def _reshape_idx(v, nw, nch):
    padn = nw * nch * K - E
    return jnp.concatenate(
        [v, jnp.full((padn,), PADROW, jnp.int32)]).reshape(nw, nch, K)


def kernel(x, edge_index, adj, s, pos,
           W_rel1, b_rel1, W_root1,
           W_rel2, b_rel2, W_root2,
           W_rel3, b_rel3, W_root3):
    src = edge_index[0]
    dst = edge_index[1]

    srcr1 = _reshape_idx(src, NW, NCH)
    dstr1 = _reshape_idx(dst, NW, NCH)
    srcr2 = _reshape_idx(src, NW2 * NPASS, NCH2H)
    dstr2 = _reshape_idx(dst, NW2 * NPASS, NCH2H)

    x16 = jnp.zeros((NP, 16), jnp.float32).at[:N, :2].set(x)
    z16 = jnp.zeros((NP, 16), jnp.float32)
    z64 = jnp.zeros((NP, DH), jnp.float32)

    A1 = jnp.zeros((16, H), jnp.float32).at[:2, :].set(W_rel1.T)
    B1 = jnp.zeros((16, H), jnp.float32).at[:2, :].set(W_root1.T)

    seg16 = _sc_segsum16(16)
    seg64 = _sc_segsum_colsplit()

    p1 = seg16(x16, srcr1, dstr1, z16)
    h1 = _tc_layer1(p1, x16, A1, B1, b_rel1.reshape(1, H))

    p2 = seg64(h1, srcr2, dstr2, z64)
    h2 = _tc_layer(p2, h1, W_rel2.T, W_root2.T, b_rel2.reshape(1, H),
                   split_out=True)

    p3 = seg64(h2, srcr2, dstr2, z64)
    h3 = _tc_layer(p3, h2, W_rel3.T, W_root3.T, b_rel3.reshape(1, H),
                   split_out=False)

    xr = h3[:N].reshape(NG, NN, H)
    xr_p = jnp.zeros((NG, 128, H), jnp.float32).at[:, :NN, :].set(xr)
    adj_p = jnp.zeros((NG, 128, 128), jnp.float32).at[:, :NN, :NN].set(adj)
    s_p = jnp.zeros((NG, 128, C), jnp.float32).at[:, :NN, :].set(s)

    out, out_adj, acc = _tc_pool(xr_p, adj_p, s_p)
    link_loss = jnp.sqrt(acc[0, 0]) / (NG * NN * NN)
    ent_loss = acc[0, 1] / (NG * NN)
    return out, out_adj, link_loss, ent_loss, pos


# speedup vs baseline: 2.1951x; 1.0235x over previous
import functools

import jax
import jax.numpy as jnp
from jax import lax
from jax.experimental import pallas as pl
from jax.experimental.pallas import tpu as pltpu
from jax.experimental.pallas import tpu_sc as plsc

N = 9996
E = 319872
NG = 147
NN = 68
H = 128
C = 16
EPS = 1e-15

NP = 10240
PADROW = NP - 1
NW = 32
K = 128
EW = (E + NW * K - 1) // (NW * K) * K
NCH = EW // K
ZR = NP // 16

DH = 64
NW2 = 16
NPASS = 2
EW2 = (E + NW2 * NPASS * K - 1) // (NW2 * NPASS * K) * K
NCH2H = EW2 // K
NCH2 = NPASS * NCH2H


def _sc_segsum16(D):
    mesh = plsc.VectorSubcoreMesh(core_axis_name="c", subcore_axis_name="s")

    @functools.partial(
        pl.kernel,
        out_type=jax.ShapeDtypeStruct((2, NP, D), jnp.float32),
        mesh=mesh,
        compiler_params=pltpu.CompilerParams(use_tc_tiling_on_sc=False),
        scratch_types=[
            pltpu.VMEM((NCH, K), jnp.int32),
            pltpu.VMEM((NCH, K), jnp.int32),
            pltpu.VMEM((2, K, D), jnp.float32),
            pltpu.VMEM_SHARED((NP, D), jnp.float32),
            pltpu.SemaphoreType.DMA,
        ],
    )
    def k(h_hbm, srcr_hbm, dstr_hbm, zeros_hbm, out_hbm,
          sidx, didx, rows_v, acc_sh, gsem):
        c = lax.axis_index("c")
        sid = lax.axis_index("s")
        w = sid * 2 + c
        pltpu.sync_copy(srcr_hbm.at[w], sidx)
        pltpu.sync_copy(dstr_hbm.at[w], didx)
        pltpu.sync_copy(zeros_hbm.at[pl.ds(sid * ZR, ZR)],
                        acc_sh.at[pl.ds(sid * ZR, ZR)])
        plsc.subcore_barrier()

        pltpu.async_copy(h_hbm.at[sidx.at[0]], rows_v.at[0], gsem)

        @pl.loop(0, NCH - 1)
        def _(j):
            pltpu.make_async_copy(h_hbm.at[sidx.at[j]],
                                  rows_v.at[j % 2], gsem).wait()
            pltpu.async_copy(h_hbm.at[sidx.at[j + 1]],
                             rows_v.at[(j + 1) % 2], gsem)
            pltpu.sync_copy(rows_v.at[j % 2], acc_sh.at[didx.at[j]],
                            add=True)

        pltpu.make_async_copy(h_hbm.at[sidx.at[NCH - 1]],
                              rows_v.at[(NCH - 1) % 2], gsem).wait()
        pltpu.sync_copy(rows_v.at[(NCH - 1) % 2],
                        acc_sh.at[didx.at[NCH - 1]], add=True)
        plsc.subcore_barrier()
        pltpu.sync_copy(acc_sh.at[pl.ds(sid * ZR, ZR)],
                        out_hbm.at[c, pl.ds(sid * ZR, ZR)])

    return k


def _sc_segsum_colsplit():
    mesh = plsc.VectorSubcoreMesh(core_axis_name="c", subcore_axis_name="s")

    @functools.partial(
        pl.kernel,
        out_type=jax.ShapeDtypeStruct((2, NP, DH), jnp.float32),
        mesh=mesh,
        compiler_params=pltpu.CompilerParams(use_tc_tiling_on_sc=False),
        scratch_types=[
            pltpu.VMEM((NCH2H, K), jnp.int32),
            pltpu.VMEM((NCH2H, K), jnp.int32),
            pltpu.VMEM((2, K, DH), jnp.float32),
            pltpu.VMEM_SHARED((NP, DH), jnp.float32),
            pltpu.VMEM_SHARED((NP, DH), jnp.float32),
            pltpu.SemaphoreType.DMA,
        ],
    )
    def k(h2_hbm, srcr_hbm, dstr_hbm, zeros_hbm, out_hbm,
          sidx, didx, rows_v, h_sh, acc_sh, gsem):
        c = lax.axis_index("c")
        sid = lax.axis_index("s")
        pltpu.sync_copy(zeros_hbm.at[pl.ds(sid * ZR, ZR)],
                        acc_sh.at[pl.ds(sid * ZR, ZR)])
        pltpu.sync_copy(h2_hbm.at[c, pl.ds(sid * ZR, ZR)],
                        h_sh.at[pl.ds(sid * ZR, ZR)])
        plsc.subcore_barrier()

        for p in range(NPASS):
            t = sid * NPASS + p
            pltpu.sync_copy(srcr_hbm.at[t], sidx)
            pltpu.sync_copy(dstr_hbm.at[t], didx)
            pltpu.async_copy(h_sh.at[sidx.at[0]], rows_v.at[0], gsem)

            @pl.loop(0, NCH2H - 1)
            def _(j):
                pltpu.make_async_copy(h_sh.at[sidx.at[j]],
                                      rows_v.at[j % 2], gsem).wait()
                pltpu.async_copy(h_sh.at[sidx.at[j + 1]],
                                 rows_v.at[(j + 1) % 2], gsem)
                pltpu.sync_copy(rows_v.at[j % 2], acc_sh.at[didx.at[j]],
                                add=True)

            pltpu.make_async_copy(h_sh.at[sidx.at[NCH2H - 1]],
                                  rows_v.at[(NCH2H - 1) % 2], gsem).wait()
            pltpu.sync_copy(rows_v.at[(NCH2H - 1) % 2],
                            acc_sh.at[didx.at[NCH2H - 1]], add=True)

        plsc.subcore_barrier()
        pltpu.sync_copy(acc_sh.at[pl.ds(sid * ZR, ZR)],
                        out_hbm.at[c, pl.ds(sid * ZR, ZR)])

    return k


BRW = 512


def _tc_layer1(p, x16, A, Br, bias):
    def body(p_ref, h_ref, a_ref, b_ref, bias_ref, o_ref):
        agg = p_ref[0] + p_ref[1]
        acc = jnp.dot(agg, a_ref[...], preferred_element_type=jnp.float32)
        acc = acc + jnp.dot(h_ref[...], b_ref[...],
                            preferred_element_type=jnp.float32)
        acc = acc + bias_ref[...]
        i = pl.program_id(0)
        rows = i * BRW + lax.broadcasted_iota(jnp.int32, (BRW, 1), 0)
        res = jnp.where(rows < N, jnp.maximum(acc, 0.0), 0.0)
        o_ref[0] = res[:, :DH]
        o_ref[1] = res[:, DH:]

    return pl.pallas_call(
        body,
        grid=(NP // BRW,),
        in_specs=[
            pl.BlockSpec((2, BRW, 16), lambda i: (0, i, 0)),
            pl.BlockSpec((BRW, 16), lambda i: (i, 0)),
            pl.BlockSpec((16, H), lambda i: (0, 0)),
            pl.BlockSpec((16, H), lambda i: (0, 0)),
            pl.BlockSpec((1, H), lambda i: (0, 0)),
        ],
        out_specs=pl.BlockSpec((2, BRW, DH), lambda i: (0, i, 0)),
        out_shape=jax.ShapeDtypeStruct((2, NP, DH), jnp.float32),
    )(p, x16, A, Br, bias)


def _tc_layer(p, h_prev, A, Br, bias, split_out):
    def body(p_ref, h_ref, a_ref, b_ref, bias_ref, o_ref):
        agg = jnp.concatenate([p_ref[0], p_ref[1]], axis=1)
        hp = jnp.concatenate([h_ref[0], h_ref[1]], axis=1)
        acc = jnp.dot(agg, a_ref[...], preferred_element_type=jnp.float32)
        acc = acc + jnp.dot(hp, b_ref[...],
                            preferred_element_type=jnp.float32)
        acc = acc + bias_ref[...]
        i = pl.program_id(0)
        rows = i * BRW + lax.broadcasted_iota(jnp.int32, (BRW, 1), 0)
        res = jnp.where(rows < N, jnp.maximum(acc, 0.0), 0.0)
        if split_out:
            o_ref[0] = res[:, :DH]
            o_ref[1] = res[:, DH:]
        else:
            o_ref[...] = res

    if split_out:
        out_spec = pl.BlockSpec((2, BRW, DH), lambda i: (0, i, 0))
        out_shape = jax.ShapeDtypeStruct((2, NP, DH), jnp.float32)
    else:
        out_spec = pl.BlockSpec((BRW, H), lambda i: (i, 0))
        out_shape = jax.ShapeDtypeStruct((NP, H), jnp.float32)

    return pl.pallas_call(
        body,
        grid=(NP // BRW,),
        in_specs=[
            pl.BlockSpec((2, BRW, DH), lambda i: (0, i, 0)),
            pl.BlockSpec((2, BRW, DH), lambda i: (0, i, 0)),
            pl.BlockSpec((H, H), lambda i: (0, 0)),
            pl.BlockSpec((H, H), lambda i: (0, 0)),
            pl.BlockSpec((1, H), lambda i: (0, 0)),
        ],
        out_specs=out_spec,
        out_shape=out_shape,
    )(p, h_prev, A, Br, bias)


BG = 7


def _tc_pool(xr, adj_p, s_p):
    def body(xr_ref, adj_ref, s_ref, out_ref, oadj_ref, acc_ref):
        g = pl.program_id(0)

        @pl.when(g == 0)
        def _():
            acc_ref[0, 0] = 0.0
            acc_ref[0, 1] = 0.0

        link_tot = jnp.float32(0.0)
        ent_tot = jnp.float32(0.0)
        rows = lax.broadcasted_iota(jnp.int32, (128, 1), 0)
        for t in range(BG):
            sg = s_ref[t]
            m = jnp.exp(sg - jnp.max(sg, axis=-1, keepdims=True))
            ssm = m / jnp.sum(m, axis=-1, keepdims=True)
            ssm = jnp.where(rows < NN, ssm, 0.0)
            xg = xr_ref[t]
            ag = adj_ref[t]
            out_ref[t] = lax.dot_general(
                ssm, xg, (((0,), (0,)), ((), ())),
                preferred_element_type=jnp.float32)
            ta = lax.dot_general(
                ssm, ag, (((0,), (0,)), ((), ())),
                preferred_element_type=jnp.float32)
            oadj_ref[t] = lax.dot_general(
                ta, ssm, (((1,), (0,)), ((), ())),
                preferred_element_type=jnp.float32)
            link = ag - lax.dot_general(
                ssm, ssm, (((1,), (1,)), ((), ())),
                preferred_element_type=jnp.float32)
            link_tot = link_tot + jnp.sum(link * link)
            ent_tot = ent_tot + jnp.sum(-ssm * jnp.log(ssm + EPS))
        acc_ref[0, 0] += link_tot
        acc_ref[0, 1] += ent_tot

    return pl.pallas_call(
        body,
        grid=(NG // BG,),
        in_specs=[
            pl.BlockSpec((BG, 128, H), lambda g: (g, 0, 0)),
            pl.BlockSpec((BG, 128, 128), lambda g: (g, 0, 0)),
            pl.BlockSpec((BG, 128, C), lambda g: (g, 0, 0)),
        ],
        out_specs=[
            pl.BlockSpec((BG, C, H), lambda g: (g, 0, 0)),
            pl.BlockSpec((BG, C, C), lambda g: (g, 0, 0)),
            pl.BlockSpec(memory_space=pltpu.SMEM),
        ],
        out_shape=[
            jax.ShapeDtypeStruct((NG, C, H), jnp.float32),
            jax.ShapeDtypeStruct((NG, C, C), jnp.float32),
            jax.ShapeDtypeStruct((1, 2), jnp.float32),
        ],
    )(xr, adj_p, s_p)


def _reshape_idx(v, nw, nch):
    padn = nw * nch * K - E
    return jnp.concatenate(
        [v, jnp.full((padn,), PADROW, jnp.int32)]).reshape(nw, nch, K)


def kernel(x, edge_index, adj, s, pos,
           W_rel1, b_rel1, W_root1,
           W_rel2, b_rel2, W_root2,
           W_rel3, b_rel3, W_root3):
    src = edge_index[0]
    dst = edge_index[1]

    srcr1 = _reshape_idx(src, NW, NCH)
    dstr1 = _reshape_idx(dst, NW, NCH)
    srcr2 = _reshape_idx(src, NW2 * NPASS, NCH2H)
    dstr2 = _reshape_idx(dst, NW2 * NPASS, NCH2H)

    x16 = jnp.zeros((NP, 16), jnp.float32).at[:N, :2].set(x)
    z16 = jnp.zeros((NP, 16), jnp.float32)
    z64 = jnp.zeros((NP, DH), jnp.float32)

    A1 = jnp.zeros((16, H), jnp.float32).at[:2, :].set(W_rel1.T)
    B1 = jnp.zeros((16, H), jnp.float32).at[:2, :].set(W_root1.T)

    seg16 = _sc_segsum16(16)
    seg64 = _sc_segsum_colsplit()

    p1 = seg16(x16, srcr1, dstr1, z16)
    h1 = _tc_layer1(p1, x16, A1, B1, b_rel1.reshape(1, H))

    p2 = seg64(h1, srcr2, dstr2, z64)
    h2 = _tc_layer(p2, h1, W_rel2.T, W_root2.T, b_rel2.reshape(1, H),
                   split_out=True)

    p3 = seg64(h2, srcr2, dstr2, z64)
    h3 = _tc_layer(p3, h2, W_rel3.T, W_root3.T, b_rel3.reshape(1, H),
                   split_out=False)

    xr = h3[:N].reshape(NG, NN, H)
    xr_p = jnp.zeros((NG, 128, H), jnp.float32).at[:, :NN, :].set(xr)
    adj_p = jnp.zeros((NG, 128, 128), jnp.float32).at[:, :NN, :NN].set(adj)
    s_p = jnp.zeros((NG, 128, C), jnp.float32).at[:, :NN, :].set(s)

    out, out_adj, acc = _tc_pool(xr_p, adj_p, s_p)
    link_loss = jnp.sqrt(acc[0, 0]) / (NG * NN * NN)
    ent_loss = acc[0, 1] / (NG * NN)
    return out, out_adj, link_loss, ent_loss, pos


# speedup vs baseline: 2.2296x; 1.0157x over previous
import functools

import jax
import jax.numpy as jnp
from jax import lax
from jax.experimental import pallas as pl
from jax.experimental.pallas import tpu as pltpu
from jax.experimental.pallas import tpu_sc as plsc

N = 9996
E = 319872
NG = 147
NN = 68
H = 128
C = 16
EPS = 1e-15

NP = 10240
PADROW = NP - 1
NW = 32
K = 128
EW = (E + NW * K - 1) // (NW * K) * K
NCH = EW // K
ZR = NP // 16

DH = 64
NW2 = 16
NPASS = 2
EW2 = (E + NW2 * NPASS * K - 1) // (NW2 * NPASS * K) * K
NCH2H = EW2 // K
NCH2 = NPASS * NCH2H


def _sc_segsum16(D):
    mesh = plsc.VectorSubcoreMesh(core_axis_name="c", subcore_axis_name="s")

    @functools.partial(
        pl.kernel,
        out_type=jax.ShapeDtypeStruct((2, NP, D), jnp.float32),
        mesh=mesh,
        compiler_params=pltpu.CompilerParams(use_tc_tiling_on_sc=False),
        scratch_types=[
            pltpu.VMEM((NCH, K), jnp.int32),
            pltpu.VMEM((NCH, K), jnp.int32),
            pltpu.VMEM((2, K, D), jnp.float32),
            pltpu.VMEM_SHARED((NP, D), jnp.float32),
            pltpu.SemaphoreType.DMA,
        ],
    )
    def k(h_hbm, srcr_hbm, dstr_hbm, zeros_hbm, out_hbm,
          sidx, didx, rows_v, acc_sh, gsem):
        c = lax.axis_index("c")
        sid = lax.axis_index("s")
        w = sid * 2 + c
        pltpu.sync_copy(srcr_hbm.at[w], sidx)
        pltpu.sync_copy(dstr_hbm.at[w], didx)
        pltpu.sync_copy(zeros_hbm.at[pl.ds(sid * ZR, ZR)],
                        acc_sh.at[pl.ds(sid * ZR, ZR)])
        plsc.subcore_barrier()

        pltpu.async_copy(h_hbm.at[sidx.at[0]], rows_v.at[0], gsem)

        @pl.loop(0, NCH - 1)
        def _(j):
            pltpu.make_async_copy(h_hbm.at[sidx.at[j]],
                                  rows_v.at[j % 2], gsem).wait()
            pltpu.async_copy(h_hbm.at[sidx.at[j + 1]],
                             rows_v.at[(j + 1) % 2], gsem)
            pltpu.sync_copy(rows_v.at[j % 2], acc_sh.at[didx.at[j]],
                            add=True)

        pltpu.make_async_copy(h_hbm.at[sidx.at[NCH - 1]],
                              rows_v.at[(NCH - 1) % 2], gsem).wait()
        pltpu.sync_copy(rows_v.at[(NCH - 1) % 2],
                        acc_sh.at[didx.at[NCH - 1]], add=True)
        plsc.subcore_barrier()
        pltpu.sync_copy(acc_sh.at[pl.ds(sid * ZR, ZR)],
                        out_hbm.at[c, pl.ds(sid * ZR, ZR)])

    return k


def _sc_segsum_colsplit():
    mesh = plsc.VectorSubcoreMesh(core_axis_name="c", subcore_axis_name="s")

    @functools.partial(
        pl.kernel,
        out_type=jax.ShapeDtypeStruct((2, NP, DH), jnp.float32),
        mesh=mesh,
        compiler_params=pltpu.CompilerParams(use_tc_tiling_on_sc=False),
        scratch_types=[
            pltpu.VMEM((NCH2H, K), jnp.int32),
            pltpu.VMEM((NCH2H, K), jnp.int32),
            pltpu.VMEM((2, K, DH), jnp.float32),
            pltpu.VMEM_SHARED((NP, DH), jnp.float32),
            pltpu.VMEM_SHARED((NP, DH), jnp.float32),
            pltpu.SemaphoreType.DMA,
        ],
    )
    def k(h2_hbm, srcr_hbm, dstr_hbm, zeros_hbm, out_hbm,
          sidx, didx, rows_v, h_sh, acc_sh, gsem):
        c = lax.axis_index("c")
        sid = lax.axis_index("s")
        pltpu.sync_copy(zeros_hbm.at[pl.ds(sid * ZR, ZR)],
                        acc_sh.at[pl.ds(sid * ZR, ZR)])
        pltpu.sync_copy(h2_hbm.at[c, pl.ds(sid * ZR, ZR)],
                        h_sh.at[pl.ds(sid * ZR, ZR)])
        plsc.subcore_barrier()

        for p in range(NPASS):
            t = sid * NPASS + p
            pltpu.sync_copy(srcr_hbm.at[t], sidx)
            pltpu.sync_copy(dstr_hbm.at[t], didx)
            pltpu.async_copy(h_sh.at[sidx.at[0]], rows_v.at[0], gsem)

            @pl.loop(0, NCH2H - 1)
            def _(j):
                pltpu.make_async_copy(h_sh.at[sidx.at[j]],
                                      rows_v.at[j % 2], gsem).wait()
                pltpu.async_copy(h_sh.at[sidx.at[j + 1]],
                                 rows_v.at[(j + 1) % 2], gsem)
                pltpu.sync_copy(rows_v.at[j % 2], acc_sh.at[didx.at[j]],
                                add=True)

            pltpu.make_async_copy(h_sh.at[sidx.at[NCH2H - 1]],
                                  rows_v.at[(NCH2H - 1) % 2], gsem).wait()
            pltpu.sync_copy(rows_v.at[(NCH2H - 1) % 2],
                            acc_sh.at[didx.at[NCH2H - 1]], add=True)

        plsc.subcore_barrier()
        pltpu.sync_copy(acc_sh.at[pl.ds(sid * ZR, ZR)],
                        out_hbm.at[c, pl.ds(sid * ZR, ZR)])

    return k


BRW = 512


def _tc_layer1(p, x16, A, Br, bias):
    def body(p_ref, h_ref, a_ref, b_ref, bias_ref, o_ref):
        agg = p_ref[0] + p_ref[1]
        acc = jnp.dot(agg, a_ref[...], preferred_element_type=jnp.float32)
        acc = acc + jnp.dot(h_ref[...], b_ref[...],
                            preferred_element_type=jnp.float32)
        acc = acc + bias_ref[...]
        i = pl.program_id(0)
        rows = i * BRW + lax.broadcasted_iota(jnp.int32, (BRW, 1), 0)
        res = jnp.where(rows < N, jnp.maximum(acc, 0.0), 0.0)
        o_ref[0] = res[:, :DH]
        o_ref[1] = res[:, DH:]

    return pl.pallas_call(
        body,
        grid=(NP // BRW,),
        in_specs=[
            pl.BlockSpec((2, BRW, 16), lambda i: (0, i, 0)),
            pl.BlockSpec((BRW, 16), lambda i: (i, 0)),
            pl.BlockSpec((16, H), lambda i: (0, 0)),
            pl.BlockSpec((16, H), lambda i: (0, 0)),
            pl.BlockSpec((1, H), lambda i: (0, 0)),
        ],
        out_specs=pl.BlockSpec((2, BRW, DH), lambda i: (0, i, 0)),
        out_shape=jax.ShapeDtypeStruct((2, NP, DH), jnp.float32),
    )(p, x16, A, Br, bias)


def _tc_layer(p, h_prev, A, Br, bias, split_out):
    def body(p_ref, h_ref, a_ref, b_ref, bias_ref, o_ref):
        agg = jnp.concatenate([p_ref[0], p_ref[1]], axis=1)
        hp = jnp.concatenate([h_ref[0], h_ref[1]], axis=1)
        acc = jnp.dot(agg, a_ref[...], preferred_element_type=jnp.float32)
        acc = acc + jnp.dot(hp, b_ref[...],
                            preferred_element_type=jnp.float32)
        acc = acc + bias_ref[...]
        i = pl.program_id(0)
        rows = i * BRW + lax.broadcasted_iota(jnp.int32, (BRW, 1), 0)
        res = jnp.where(rows < N, jnp.maximum(acc, 0.0), 0.0)
        if split_out:
            o_ref[0] = res[:, :DH]
            o_ref[1] = res[:, DH:]
        else:
            o_ref[...] = res

    if split_out:
        out_spec = pl.BlockSpec((2, BRW, DH), lambda i: (0, i, 0))
        out_shape = jax.ShapeDtypeStruct((2, NP, DH), jnp.float32)
    else:
        out_spec = pl.BlockSpec((BRW, H), lambda i: (i, 0))
        out_shape = jax.ShapeDtypeStruct((NP, H), jnp.float32)

    return pl.pallas_call(
        body,
        grid=(NP // BRW,),
        in_specs=[
            pl.BlockSpec((2, BRW, DH), lambda i: (0, i, 0)),
            pl.BlockSpec((2, BRW, DH), lambda i: (0, i, 0)),
            pl.BlockSpec((H, H), lambda i: (0, 0)),
            pl.BlockSpec((H, H), lambda i: (0, 0)),
            pl.BlockSpec((1, H), lambda i: (0, 0)),
        ],
        out_specs=out_spec,
        out_shape=out_shape,
    )(p, h_prev, A, Br, bias)


BG = 7


def _tc_pool(h3, adj, s):
    def body(xr_ref, adj_ref, s_ref, out_ref, oadj_ref, acc_ref):
        g = pl.program_id(0)

        @pl.when(g == 0)
        def _():
            acc_ref[0, 0] = 0.0
            acc_ref[0, 1] = 0.0

        link_tot = jnp.float32(0.0)
        ent_tot = jnp.float32(0.0)
        for t in range(BG):
            sg = s_ref[t]
            m = jnp.exp(sg - jnp.max(sg, axis=-1, keepdims=True))
            ssm = m / jnp.sum(m, axis=-1, keepdims=True)
            xg = xr_ref[t]
            ag = adj_ref[t]
            out_ref[t] = lax.dot_general(
                ssm, xg, (((0,), (0,)), ((), ())),
                preferred_element_type=jnp.float32)
            ta = lax.dot_general(
                ssm, ag, (((0,), (0,)), ((), ())),
                preferred_element_type=jnp.float32)
            oadj_ref[t] = lax.dot_general(
                ta, ssm, (((1,), (0,)), ((), ())),
                preferred_element_type=jnp.float32)
            link = ag - lax.dot_general(
                ssm, ssm, (((1,), (1,)), ((), ())),
                preferred_element_type=jnp.float32)
            link_tot = link_tot + jnp.sum(link * link)
            ent_tot = ent_tot + jnp.sum(-ssm * jnp.log(ssm + EPS))
        acc_ref[0, 0] += link_tot
        acc_ref[0, 1] += ent_tot

    return pl.pallas_call(
        body,
        grid=(NG // BG,),
        in_specs=[
            pl.BlockSpec((BG, NN, H), lambda g: (g, 0, 0)),
            pl.BlockSpec((BG, NN, NN), lambda g: (g, 0, 0)),
            pl.BlockSpec((BG, NN, C), lambda g: (g, 0, 0)),
        ],
        out_specs=[
            pl.BlockSpec((BG, C, H), lambda g: (g, 0, 0)),
            pl.BlockSpec((BG, C, C), lambda g: (g, 0, 0)),
            pl.BlockSpec(memory_space=pltpu.SMEM),
        ],
        out_shape=[
            jax.ShapeDtypeStruct((NG, C, H), jnp.float32),
            jax.ShapeDtypeStruct((NG, C, C), jnp.float32),
            jax.ShapeDtypeStruct((1, 2), jnp.float32),
        ],
    )(h3, adj, s)


def _reshape_idx(v, nw, nch):
    padn = nw * nch * K - E
    return jnp.concatenate(
        [v, jnp.full((padn,), PADROW, jnp.int32)]).reshape(nw, nch, K)


def kernel(x, edge_index, adj, s, pos,
           W_rel1, b_rel1, W_root1,
           W_rel2, b_rel2, W_root2,
           W_rel3, b_rel3, W_root3):
    src = edge_index[0]
    dst = edge_index[1]

    srcr = _reshape_idx(src, NW, NCH)
    dstr = _reshape_idx(dst, NW, NCH)

    x16 = jnp.zeros((NP, 16), jnp.float32).at[:N, :2].set(x)
    z16 = jnp.zeros((NP, 16), jnp.float32)
    z64 = jnp.zeros((NP, DH), jnp.float32)

    A1 = jnp.zeros((16, H), jnp.float32).at[:2, :].set(W_rel1.T)
    B1 = jnp.zeros((16, H), jnp.float32).at[:2, :].set(W_root1.T)

    seg16 = _sc_segsum16(16)
    seg64 = _sc_segsum_colsplit()

    p1 = seg16(x16, srcr, dstr, z16)
    h1 = _tc_layer1(p1, x16, A1, B1, b_rel1.reshape(1, H))

    p2 = seg64(h1, srcr, dstr, z64)
    h2 = _tc_layer(p2, h1, W_rel2.T, W_root2.T, b_rel2.reshape(1, H),
                   split_out=True)

    p3 = seg64(h2, srcr, dstr, z64)
    h3 = _tc_layer(p3, h2, W_rel3.T, W_root3.T, b_rel3.reshape(1, H),
                   split_out=False)

    xr = h3[:N].reshape(NG, NN, H)
    out, out_adj, acc = _tc_pool(xr, adj, s)
    link_loss = jnp.sqrt(acc[0, 0]) / (NG * NN * NN)
    ent_loss = acc[0, 1] / (NG * NN)
    return out, out_adj, link_loss, ent_loss, pos


# speedup vs baseline: 2.2961x; 1.0298x over previous
import functools

import jax
import jax.numpy as jnp
from jax import lax
from jax.experimental import pallas as pl
from jax.experimental.pallas import tpu as pltpu
from jax.experimental.pallas import tpu_sc as plsc

N = 9996
E = 319872
NG = 147
NN = 68
H = 128
C = 16
EPS = 1e-15

NP = 10240
PADROW = NP - 1
NW = 32
K = 128
EW = (E + NW * K - 1) // (NW * K) * K
NCH = EW // K
ZR = NP // 16

DH = 64
NW2 = 16
NPASS = 2
EW2 = (E + NW2 * NPASS * K - 1) // (NW2 * NPASS * K) * K
NCH2H = EW2 // K
NCH2 = NPASS * NCH2H


def _sc_segsum16(D):
    mesh = plsc.VectorSubcoreMesh(core_axis_name="c", subcore_axis_name="s")

    @functools.partial(
        pl.kernel,
        out_type=jax.ShapeDtypeStruct((2, NP, D), jnp.float32),
        mesh=mesh,
        compiler_params=pltpu.CompilerParams(use_tc_tiling_on_sc=False),
        scratch_types=[
            pltpu.VMEM((NCH, K), jnp.int32),
            pltpu.VMEM((NCH, K), jnp.int32),
            pltpu.VMEM((3, K, D), jnp.float32),
            pltpu.VMEM_SHARED((NP, D), jnp.float32),
            pltpu.SemaphoreType.DMA,
        ],
    )
    def k(h_hbm, srcr_hbm, dstr_hbm, zeros_hbm, out_hbm,
          sidx, didx, rows_v, acc_sh, gsem):
        c = lax.axis_index("c")
        sid = lax.axis_index("s")
        w = sid * 2 + c
        pltpu.sync_copy(srcr_hbm.at[w], sidx)
        pltpu.sync_copy(dstr_hbm.at[w], didx)
        pltpu.sync_copy(zeros_hbm.at[pl.ds(sid * ZR, ZR)],
                        acc_sh.at[pl.ds(sid * ZR, ZR)])
        plsc.subcore_barrier()

        pltpu.async_copy(h_hbm.at[sidx.at[0]], rows_v.at[0], gsem)
        pltpu.async_copy(h_hbm.at[sidx.at[1]], rows_v.at[1], gsem)

        @pl.loop(0, NCH - 2)
        def _(j):
            pltpu.make_async_copy(h_hbm.at[sidx.at[j]],
                                  rows_v.at[j % 3], gsem).wait()
            pltpu.async_copy(h_hbm.at[sidx.at[j + 2]],
                             rows_v.at[(j + 2) % 3], gsem)
            pltpu.sync_copy(rows_v.at[j % 3], acc_sh.at[didx.at[j]],
                            add=True)

        @pl.loop(NCH - 2, NCH)
        def _(j):
            pltpu.make_async_copy(h_hbm.at[sidx.at[j]],
                                  rows_v.at[j % 3], gsem).wait()
            pltpu.sync_copy(rows_v.at[j % 3], acc_sh.at[didx.at[j]],
                            add=True)

        plsc.subcore_barrier()
        pltpu.sync_copy(acc_sh.at[pl.ds(sid * ZR, ZR)],
                        out_hbm.at[c, pl.ds(sid * ZR, ZR)])

    return k


def _sc_segsum_colsplit():
    mesh = plsc.VectorSubcoreMesh(core_axis_name="c", subcore_axis_name="s")

    @functools.partial(
        pl.kernel,
        out_type=jax.ShapeDtypeStruct((2, NP, DH), jnp.float32),
        mesh=mesh,
        compiler_params=pltpu.CompilerParams(use_tc_tiling_on_sc=False),
        scratch_types=[
            pltpu.VMEM((NCH2H, K), jnp.int32),
            pltpu.VMEM((NCH2H, K), jnp.int32),
            pltpu.VMEM((3, K, DH), jnp.float32),
            pltpu.VMEM_SHARED((NP, DH), jnp.float32),
            pltpu.VMEM_SHARED((NP, DH), jnp.float32),
            pltpu.SemaphoreType.DMA,
        ],
    )
    def k(h2_hbm, srcr_hbm, dstr_hbm, zeros_hbm, out_hbm,
          sidx, didx, rows_v, h_sh, acc_sh, gsem):
        c = lax.axis_index("c")
        sid = lax.axis_index("s")
        pltpu.sync_copy(zeros_hbm.at[pl.ds(sid * ZR, ZR)],
                        acc_sh.at[pl.ds(sid * ZR, ZR)])
        pltpu.sync_copy(h2_hbm.at[c, pl.ds(sid * ZR, ZR)],
                        h_sh.at[pl.ds(sid * ZR, ZR)])
        plsc.subcore_barrier()

        for p in range(NPASS):
            t = sid * NPASS + p
            pltpu.sync_copy(srcr_hbm.at[t], sidx)
            pltpu.sync_copy(dstr_hbm.at[t], didx)
            pltpu.async_copy(h_sh.at[sidx.at[0]], rows_v.at[0], gsem)
            pltpu.async_copy(h_sh.at[sidx.at[1]], rows_v.at[1], gsem)

            @pl.loop(0, NCH2H - 2)
            def _(j):
                pltpu.make_async_copy(h_sh.at[sidx.at[j]],
                                      rows_v.at[j % 3], gsem).wait()
                pltpu.async_copy(h_sh.at[sidx.at[j + 2]],
                                 rows_v.at[(j + 2) % 3], gsem)
                pltpu.sync_copy(rows_v.at[j % 3], acc_sh.at[didx.at[j]],
                                add=True)

            @pl.loop(NCH2H - 2, NCH2H)
            def _(j):
                pltpu.make_async_copy(h_sh.at[sidx.at[j]],
                                      rows_v.at[j % 3], gsem).wait()
                pltpu.sync_copy(rows_v.at[j % 3], acc_sh.at[didx.at[j]],
                                add=True)

        plsc.subcore_barrier()
        pltpu.sync_copy(acc_sh.at[pl.ds(sid * ZR, ZR)],
                        out_hbm.at[c, pl.ds(sid * ZR, ZR)])

    return k


BRW = 512


def _tc_layer1(p, x16, A, Br, bias):
    def body(p_ref, h_ref, a_ref, b_ref, bias_ref, o_ref):
        agg = p_ref[0] + p_ref[1]
        acc = jnp.dot(agg, a_ref[...], preferred_element_type=jnp.float32)
        acc = acc + jnp.dot(h_ref[...], b_ref[...],
                            preferred_element_type=jnp.float32)
        acc = acc + bias_ref[...]
        i = pl.program_id(0)
        rows = i * BRW + lax.broadcasted_iota(jnp.int32, (BRW, 1), 0)
        res = jnp.where(rows < N, jnp.maximum(acc, 0.0), 0.0)
        o_ref[0] = res[:, :DH]
        o_ref[1] = res[:, DH:]

    return pl.pallas_call(
        body,
        grid=(NP // BRW,),
        in_specs=[
            pl.BlockSpec((2, BRW, 16), lambda i: (0, i, 0)),
            pl.BlockSpec((BRW, 16), lambda i: (i, 0)),
            pl.BlockSpec((16, H), lambda i: (0, 0)),
            pl.BlockSpec((16, H), lambda i: (0, 0)),
            pl.BlockSpec((1, H), lambda i: (0, 0)),
        ],
        out_specs=pl.BlockSpec((2, BRW, DH), lambda i: (0, i, 0)),
        out_shape=jax.ShapeDtypeStruct((2, NP, DH), jnp.float32),
    )(p, x16, A, Br, bias)


def _tc_layer(p, h_prev, A, Br, bias, split_out):
    def body(p_ref, h_ref, a_ref, b_ref, bias_ref, o_ref):
        agg = jnp.concatenate([p_ref[0], p_ref[1]], axis=1)
        hp = jnp.concatenate([h_ref[0], h_ref[1]], axis=1)
        acc = jnp.dot(agg, a_ref[...], preferred_element_type=jnp.float32)
        acc = acc + jnp.dot(hp, b_ref[...],
                            preferred_element_type=jnp.float32)
        acc = acc + bias_ref[...]
        i = pl.program_id(0)
        rows = i * BRW + lax.broadcasted_iota(jnp.int32, (BRW, 1), 0)
        res = jnp.where(rows < N, jnp.maximum(acc, 0.0), 0.0)
        if split_out:
            o_ref[0] = res[:, :DH]
            o_ref[1] = res[:, DH:]
        else:
            o_ref[...] = res

    if split_out:
        out_spec = pl.BlockSpec((2, BRW, DH), lambda i: (0, i, 0))
        out_shape = jax.ShapeDtypeStruct((2, NP, DH), jnp.float32)
    else:
        out_spec = pl.BlockSpec((BRW, H), lambda i: (i, 0))
        out_shape = jax.ShapeDtypeStruct((NP, H), jnp.float32)

    return pl.pallas_call(
        body,
        grid=(NP // BRW,),
        in_specs=[
            pl.BlockSpec((2, BRW, DH), lambda i: (0, i, 0)),
            pl.BlockSpec((2, BRW, DH), lambda i: (0, i, 0)),
            pl.BlockSpec((H, H), lambda i: (0, 0)),
            pl.BlockSpec((H, H), lambda i: (0, 0)),
            pl.BlockSpec((1, H), lambda i: (0, 0)),
        ],
        out_specs=out_spec,
        out_shape=out_shape,
    )(p, h_prev, A, Br, bias)


BG = 7


def _tc_pool(h3, adj, s):
    def body(xr_ref, adj_ref, s_ref, out_ref, oadj_ref, acc_ref):
        g = pl.program_id(0)

        @pl.when(g == 0)
        def _():
            acc_ref[0, 0] = 0.0
            acc_ref[0, 1] = 0.0

        link_tot = jnp.float32(0.0)
        ent_tot = jnp.float32(0.0)
        for t in range(BG):
            sg = s_ref[t]
            m = jnp.exp(sg - jnp.max(sg, axis=-1, keepdims=True))
            ssm = m / jnp.sum(m, axis=-1, keepdims=True)
            xg = xr_ref[t]
            ag = adj_ref[t]
            out_ref[t] = lax.dot_general(
                ssm, xg, (((0,), (0,)), ((), ())),
                preferred_element_type=jnp.float32)
            ta = lax.dot_general(
                ssm, ag, (((0,), (0,)), ((), ())),
                preferred_element_type=jnp.float32)
            oadj_ref[t] = lax.dot_general(
                ta, ssm, (((1,), (0,)), ((), ())),
                preferred_element_type=jnp.float32)
            link = ag - lax.dot_general(
                ssm, ssm, (((1,), (1,)), ((), ())),
                preferred_element_type=jnp.float32)
            link_tot = link_tot + jnp.sum(link * link)
            ent_tot = ent_tot + jnp.sum(-ssm * jnp.log(ssm + EPS))
        acc_ref[0, 0] += link_tot
        acc_ref[0, 1] += ent_tot

    return pl.pallas_call(
        body,
        grid=(NG // BG,),
        in_specs=[
            pl.BlockSpec((BG, NN, H), lambda g: (g, 0, 0)),
            pl.BlockSpec((BG, NN, NN), lambda g: (g, 0, 0)),
            pl.BlockSpec((BG, NN, C), lambda g: (g, 0, 0)),
        ],
        out_specs=[
            pl.BlockSpec((BG, C, H), lambda g: (g, 0, 0)),
            pl.BlockSpec((BG, C, C), lambda g: (g, 0, 0)),
            pl.BlockSpec(memory_space=pltpu.SMEM),
        ],
        out_shape=[
            jax.ShapeDtypeStruct((NG, C, H), jnp.float32),
            jax.ShapeDtypeStruct((NG, C, C), jnp.float32),
            jax.ShapeDtypeStruct((1, 2), jnp.float32),
        ],
    )(h3, adj, s)


def _reshape_idx(v, nw, nch):
    padn = nw * nch * K - E
    return jnp.concatenate(
        [v, jnp.full((padn,), PADROW, jnp.int32)]).reshape(nw, nch, K)


def kernel(x, edge_index, adj, s, pos,
           W_rel1, b_rel1, W_root1,
           W_rel2, b_rel2, W_root2,
           W_rel3, b_rel3, W_root3):
    src = edge_index[0]
    dst = edge_index[1]

    srcr = _reshape_idx(src, NW, NCH)
    dstr = _reshape_idx(dst, NW, NCH)

    x16 = jnp.zeros((NP, 16), jnp.float32).at[:N, :2].set(x)
    z16 = jnp.zeros((NP, 16), jnp.float32)
    z64 = jnp.zeros((NP, DH), jnp.float32)

    A1 = jnp.zeros((16, H), jnp.float32).at[:2, :].set(W_rel1.T)
    B1 = jnp.zeros((16, H), jnp.float32).at[:2, :].set(W_root1.T)

    seg16 = _sc_segsum16(16)
    seg64 = _sc_segsum_colsplit()

    p1 = seg16(x16, srcr, dstr, z16)
    h1 = _tc_layer1(p1, x16, A1, B1, b_rel1.reshape(1, H))

    p2 = seg64(h1, srcr, dstr, z64)
    h2 = _tc_layer(p2, h1, W_rel2.T, W_root2.T, b_rel2.reshape(1, H),
                   split_out=True)

    p3 = seg64(h2, srcr, dstr, z64)
    h3 = _tc_layer(p3, h2, W_rel3.T, W_root3.T, b_rel3.reshape(1, H),
                   split_out=False)

    xr = h3[:N].reshape(NG, NN, H)
    out, out_adj, acc = _tc_pool(xr, adj, s)
    link_loss = jnp.sqrt(acc[0, 0]) / (NG * NN * NN)
    ent_loss = acc[0, 1] / (NG * NN)
    return out, out_adj, link_loss, ent_loss, pos
